# baseline (speedup 1.0000x reference)
import functools

import jax
import jax.numpy as jnp
from jax.experimental import pallas as pl
from jax.experimental.pallas import tpu as pltpu

S = 2048
H = 2048
F = 4096
E = 8
K = 2
T = 128
A = S * K
NT = A // T + E
PAD = NT * T
FC = 256
NF = F // FC


def _ffn_kernel(tile_e_ref, x_ref, w1_ref, w3_ref, w2_ref, ws_ref, out_ref):
    fc = pl.program_id(0)
    t = pl.program_id(1)
    x = x_ref[...]
    w1b = w1_ref[0]
    w3b = w3_ref[0]
    w2b = w2_ref[0]
    h1 = jnp.dot(x, w1b, preferred_element_type=jnp.float32)
    h3 = jnp.dot(x, w3b, preferred_element_type=jnp.float32)
    g = h1 * jax.lax.logistic(h1) * h3
    part = jnp.dot(g, w2b, preferred_element_type=jnp.float32)
    part = part * ws_ref[:, :1]
    row = t * T

    @pl.when(fc == 0)
    def _init():
        out_ref[pl.ds(row, T), :] = part

    @pl.when(fc != 0)
    def _acc():
        out_ref[pl.ds(row, T), :] += part


def _grouped_ffn(tile_e, x_pad, w1, w3, w2, wscale):
    grid_spec = pltpu.PrefetchScalarGridSpec(
        num_scalar_prefetch=1,
        grid=(NF, NT),
        in_specs=[
            pl.BlockSpec((T, H), lambda fc, t, te: (t, 0)),
            pl.BlockSpec((1, H, FC), lambda fc, t, te: (te[t], 0, fc)),
            pl.BlockSpec((1, H, FC), lambda fc, t, te: (te[t], 0, fc)),
            pl.BlockSpec((1, FC, H), lambda fc, t, te: (te[t], fc, 0)),
            pl.BlockSpec((T, 128), lambda fc, t, te: (t, 0)),
        ],
        out_specs=pl.BlockSpec((PAD, H), lambda fc, t, te: (0, 0)),
    )
    return pl.pallas_call(
        _ffn_kernel,
        grid_spec=grid_spec,
        out_shape=jax.ShapeDtypeStruct((PAD, H), jnp.float32),
    )(tile_e, x_pad, w1, w3, w2, wscale)


def kernel(hidden_states, gate_w, w1, w3, w2):
    router_logits = jnp.einsum('bsh,he->bse', hidden_states, gate_w).astype(jnp.float32)
    routing_weights, selected_experts = jax.lax.top_k(router_logits, k=K)
    routing_weights = jax.nn.softmax(routing_weights.astype(jnp.float32), axis=-1)

    x = hidden_states.reshape(S, H)
    flat_e = selected_experts.reshape(A).astype(jnp.int32)
    flat_w = routing_weights.reshape(A)

    order = jnp.argsort(flat_e)
    se = flat_e[order]
    stok = (order // K).astype(jnp.int32)
    sw = flat_w[order]
    counts = jnp.bincount(flat_e, length=E)
    offs = jnp.concatenate([jnp.zeros((1,), counts.dtype), jnp.cumsum(counts)[:-1]])
    ntiles_e = (counts + T - 1) // T
    tile_off = jnp.concatenate([jnp.zeros((1,), counts.dtype), jnp.cumsum(ntiles_e)[:-1]])
    total_tiles = tile_off[E - 1] + ntiles_e[E - 1]

    s_idx = jnp.arange(NT)
    e_of_tile = (jnp.searchsorted(tile_off, s_idx, side='right') - 1).astype(jnp.int32)
    e_of_tile = jnp.clip(e_of_tile, 0, E - 1)
    local = s_idx - tile_off[e_of_tile]
    row_start = offs[e_of_tile] + local * T

    j = jnp.arange(T)
    q_mat = row_start[:, None] + j[None, :]
    valid = ((local * T)[:, None] + j[None, :] < counts[e_of_tile][:, None])
    valid = valid & (s_idx < total_tiles)[:, None]
    q_clip = jnp.clip(q_mat, 0, A - 1)
    tok_pad = jnp.where(valid, stok[q_clip], 0).reshape(PAD).astype(jnp.int32)
    w_pad = jnp.where(valid, sw[q_clip], 0.0).reshape(PAD)
    tile_e = jnp.where(s_idx < total_tiles, e_of_tile, E - 1).astype(jnp.int32)

    qq = jnp.arange(A)
    r = qq - offs[se]
    p_of_q = ((tile_off[se] + r // T) * T + r % T).astype(jnp.int32)
    p_flat = jnp.zeros((A,), jnp.int32).at[order].set(p_of_q)
    p0 = p_flat[0::2]
    p1 = p_flat[1::2]

    x_pad = jnp.take(x, tok_pad, axis=0)

    wscale = jnp.broadcast_to(w_pad[:, None], (PAD, 128))

    y_pad = _grouped_ffn(tile_e, x_pad, w1, w3, w2, wscale)

    out = jnp.take(y_pad, p0, axis=0) + jnp.take(y_pad, p1, axis=0)

    return (out.reshape(1, S, H), router_logits)


# speedup vs baseline: 1.4024x; 1.4024x over previous
import functools

import jax
import jax.numpy as jnp
from jax.experimental import pallas as pl
from jax.experimental.pallas import tpu as pltpu

S = 2048
H = 2048
F = 4096
E = 8
K = 2
T = 256
A = S * K
NT = A // T + E
PAD = NT * T
FC = 512
NF = F // FC


def _ffn_kernel(tile_e_ref, x_ref, w1_ref, w3_ref, w2_ref, out_ref):
    fc = pl.program_id(0)
    t = pl.program_id(1)
    x = x_ref[...]
    w1b = w1_ref[0]
    w3b = w3_ref[0]
    w2b = w2_ref[0]
    h1 = jnp.dot(x, w1b, preferred_element_type=jnp.float32)
    h3 = jnp.dot(x, w3b, preferred_element_type=jnp.float32)
    g = h1 * jax.lax.logistic(h1) * h3
    part = jnp.dot(g, w2b, preferred_element_type=jnp.float32)
    row = t * T

    @pl.when(fc == 0)
    def _init():
        out_ref[pl.ds(row, T), :] = part.astype(jnp.bfloat16)

    @pl.when(fc != 0)
    def _acc():
        acc = out_ref[pl.ds(row, T), :].astype(jnp.float32) + part
        out_ref[pl.ds(row, T), :] = acc.astype(jnp.bfloat16)


def _grouped_ffn(tile_e, x_pad, w1, w3, w2):
    grid_spec = pltpu.PrefetchScalarGridSpec(
        num_scalar_prefetch=1,
        grid=(NF, NT),
        in_specs=[
            pl.BlockSpec((T, H), lambda fc, t, te: (t, 0)),
            pl.BlockSpec((1, H, FC), lambda fc, t, te: (te[t], 0, fc)),
            pl.BlockSpec((1, H, FC), lambda fc, t, te: (te[t], 0, fc)),
            pl.BlockSpec((1, FC, H), lambda fc, t, te: (te[t], fc, 0)),
        ],
        out_specs=pl.BlockSpec((PAD, H), lambda fc, t, te: (0, 0)),
    )
    return pl.pallas_call(
        _ffn_kernel,
        grid_spec=grid_spec,
        out_shape=jax.ShapeDtypeStruct((PAD, H), jnp.bfloat16),
    )(tile_e, x_pad, w1, w3, w2)


def kernel(hidden_states, gate_w, w1, w3, w2):
    router_logits = jnp.einsum('bsh,he->bse', hidden_states, gate_w).astype(jnp.float32)
    routing_weights, selected_experts = jax.lax.top_k(router_logits, k=K)
    routing_weights = jax.nn.softmax(routing_weights.astype(jnp.float32), axis=-1)

    x = hidden_states.reshape(S, H)
    flat_e = selected_experts.reshape(A).astype(jnp.int32)
    flat_w = routing_weights.reshape(A)

    order = jnp.argsort(flat_e)
    se = flat_e[order]
    stok = (order // K).astype(jnp.int32)
    sw = flat_w[order]
    counts = jnp.bincount(flat_e, length=E)
    offs = jnp.concatenate([jnp.zeros((1,), counts.dtype), jnp.cumsum(counts)[:-1]])
    ntiles_e = (counts + T - 1) // T
    tile_off = jnp.concatenate([jnp.zeros((1,), counts.dtype), jnp.cumsum(ntiles_e)[:-1]])
    total_tiles = tile_off[E - 1] + ntiles_e[E - 1]

    s_idx = jnp.arange(NT)
    e_of_tile = (jnp.searchsorted(tile_off, s_idx, side='right') - 1).astype(jnp.int32)
    e_of_tile = jnp.clip(e_of_tile, 0, E - 1)
    local = s_idx - tile_off[e_of_tile]
    row_start = offs[e_of_tile] + local * T

    j = jnp.arange(T)
    q_mat = row_start[:, None] + j[None, :]
    valid = ((local * T)[:, None] + j[None, :] < counts[e_of_tile][:, None])
    valid = valid & (s_idx < total_tiles)[:, None]
    q_clip = jnp.clip(q_mat, 0, A - 1)
    tok_pad = jnp.where(valid, stok[q_clip], 0).reshape(PAD).astype(jnp.int32)
    w_pad = jnp.where(valid, sw[q_clip], 0.0).reshape(PAD)
    tile_e = jnp.where(s_idx < total_tiles, e_of_tile, E - 1).astype(jnp.int32)

    qq = jnp.arange(A)
    r = qq - offs[se]
    p_of_q = ((tile_off[se] + r // T) * T + r % T).astype(jnp.int32)
    p_flat = jnp.zeros((A,), jnp.int32).at[order].set(p_of_q)
    p0 = p_flat[0::2]
    p1 = p_flat[1::2]

    x_pad = jnp.take(x, tok_pad, axis=0)

    y_pad = _grouped_ffn(tile_e, x_pad, w1, w3, w2)

    w0 = flat_w[0::2][:, None]
    w1c = flat_w[1::2][:, None]
    out = (w0 * jnp.take(y_pad, p0, axis=0).astype(jnp.float32)
           + w1c * jnp.take(y_pad, p1, axis=0).astype(jnp.float32))

    return (out.reshape(1, S, H), router_logits)
